# baseline (speedup 1.0000x reference)
import jax
import jax.numpy as jnp
from jax.experimental import pallas as pl
from jax.experimental.pallas import tpu as pltpu

_DEPTH = 2


def _chunk_table(B, N):
    table = []
    for b in range(B):
        if b == 0:
            sizes = [N // 8, N // 8, N // 4, N // 2]
        elif b == B - 1:
            sizes = [N // 2, N // 4, N // 8, N // 8]
        else:
            sizes = [N // 2, N // 2]
        s = 0
        for rows in sizes:
            table.append((b * N + s, s, rows))
            s += rows
    return table


def _body_factory(B, N, D):
    table = _chunk_table(B, N)
    pos_chunks = [(ps, rows) for (fs, ps, rows) in table if fs < N]

    def body(x_hbm, pos_hbm, o_hbm, x_buf, o_buf, pos_vmem, in_sems, out_sems, pos_sems):
        def in_cp(i, k):
            fs, _, rows = table[i]
            return pltpu.make_async_copy(
                x_hbm.at[pl.ds(fs, rows)], x_buf.at[k, pl.ds(0, rows)],
                in_sems.at[k])

        def out_cp(i, k):
            fs, _, rows = table[i]
            return pltpu.make_async_copy(
                o_buf.at[k, pl.ds(0, rows)], o_hbm.at[pl.ds(fs, rows)],
                out_sems.at[k])

        def pos_cp(j):
            ps, rows = pos_chunks[j]
            return pltpu.make_async_copy(
                pos_hbm.at[pl.ds(ps, rows)], pos_vmem.at[pl.ds(ps, rows)],
                pos_sems.at[j])

        pos_cp(0).start()
        for k in range(_DEPTH):
            in_cp(k, k).start()
        for j in range(1, len(pos_chunks)):
            pos_cp(j).start()

        n_chunks = len(table)
        for i in range(n_chunks):
            k = i % _DEPTH
            _, ps, rows = table[i]
            if i < len(pos_chunks):
                pos_cp(i).wait()
            in_cp(i, k).wait()
            if i >= _DEPTH:
                out_cp(i - _DEPTH, k).wait()
            o_buf[k, pl.ds(0, rows)] = (
                x_buf[k, pl.ds(0, rows)] + pos_vmem[pl.ds(ps, rows), :]
            )
            out_cp(i, k).start()
            if i + _DEPTH < n_chunks:
                in_cp(i + _DEPTH, k).start()

        for i in range(n_chunks - _DEPTH, n_chunks):
            out_cp(i, i % _DEPTH).wait()

    return body


@jax.jit
def kernel(x, token_embedding):
    B, N, D = x.shape
    out = pl.pallas_call(
        _body_factory(B, N, D),
        in_specs=[
            pl.BlockSpec(memory_space=pltpu.HBM),
            pl.BlockSpec(memory_space=pltpu.HBM),
        ],
        out_specs=pl.BlockSpec(memory_space=pltpu.HBM),
        out_shape=jax.ShapeDtypeStruct((B * N, D), x.dtype),
        scratch_shapes=[
            pltpu.VMEM((_DEPTH, N // 2, D), x.dtype),
            pltpu.VMEM((_DEPTH, N // 2, D), x.dtype),
            pltpu.VMEM((N, D), x.dtype),
            pltpu.SemaphoreType.DMA((_DEPTH,)),
            pltpu.SemaphoreType.DMA((_DEPTH,)),
            pltpu.SemaphoreType.DMA((sum(1 for (fs, _, _) in _chunk_table(B, N) if fs < N),)),
        ],
    )(x.reshape(B * N, D), token_embedding)
    return out.reshape(B, N, D)


# speedup vs baseline: 1.0083x; 1.0083x over previous
import jax
import jax.numpy as jnp
from jax.experimental import pallas as pl
from jax.experimental.pallas import tpu as pltpu

_DEPTH = 2


def _chunk_table(B, N):
    table = []
    for b in range(B):
        if b == 0:
            sizes = [N // 4, N // 4, N // 2]
        elif b == B - 1:
            sizes = [N // 2, N // 4, N // 4]
        else:
            sizes = [N // 2, N // 2]
        s = 0
        for rows in sizes:
            table.append((b * N + s, s, rows))
            s += rows
    return table


def _body_factory(B, N, D):
    table = _chunk_table(B, N)
    pos_chunks = [(ps, rows) for (fs, ps, rows) in table if fs < N]

    def body(x_hbm, pos_hbm, o_hbm, x_buf, o_buf, pos_vmem, in_sems, out_sems, pos_sems):
        def in_cp(i, k):
            fs, _, rows = table[i]
            return pltpu.make_async_copy(
                x_hbm.at[pl.ds(fs, rows)], x_buf.at[k, pl.ds(0, rows)],
                in_sems.at[k])

        def out_cp(i, k):
            fs, _, rows = table[i]
            return pltpu.make_async_copy(
                o_buf.at[k, pl.ds(0, rows)], o_hbm.at[pl.ds(fs, rows)],
                out_sems.at[k])

        def pos_cp(j):
            ps, rows = pos_chunks[j]
            return pltpu.make_async_copy(
                pos_hbm.at[pl.ds(ps, rows)], pos_vmem.at[pl.ds(ps, rows)],
                pos_sems.at[j])

        pos_cp(0).start()
        for k in range(_DEPTH):
            in_cp(k, k).start()
        for j in range(1, len(pos_chunks)):
            pos_cp(j).start()

        n_chunks = len(table)
        for i in range(n_chunks):
            k = i % _DEPTH
            _, ps, rows = table[i]
            if i < len(pos_chunks):
                pos_cp(i).wait()
            in_cp(i, k).wait()
            if i >= _DEPTH:
                out_cp(i - _DEPTH, k).wait()
            o_buf[k, pl.ds(0, rows)] = (
                x_buf[k, pl.ds(0, rows)] + pos_vmem[pl.ds(ps, rows), :]
            )
            out_cp(i, k).start()
            if i + _DEPTH < n_chunks:
                in_cp(i + _DEPTH, k).start()

        for i in range(n_chunks - _DEPTH, n_chunks):
            out_cp(i, i % _DEPTH).wait()

    return body


@jax.jit
def kernel(x, token_embedding):
    B, N, D = x.shape
    out = pl.pallas_call(
        _body_factory(B, N, D),
        in_specs=[
            pl.BlockSpec(memory_space=pltpu.HBM),
            pl.BlockSpec(memory_space=pltpu.HBM),
        ],
        out_specs=pl.BlockSpec(memory_space=pltpu.HBM),
        out_shape=jax.ShapeDtypeStruct((B * N, D), x.dtype),
        scratch_shapes=[
            pltpu.VMEM((_DEPTH, N // 2, D), x.dtype),
            pltpu.VMEM((_DEPTH, N // 2, D), x.dtype),
            pltpu.VMEM((N, D), x.dtype),
            pltpu.SemaphoreType.DMA((_DEPTH,)),
            pltpu.SemaphoreType.DMA((_DEPTH,)),
            pltpu.SemaphoreType.DMA((sum(1 for (fs, _, _) in _chunk_table(B, N) if fs < N),)),
        ],
    )(x.reshape(B * N, D), token_embedding)
    return out.reshape(B, N, D)


# speedup vs baseline: 1.0095x; 1.0012x over previous
import jax
import jax.numpy as jnp
from jax.experimental import pallas as pl
from jax.experimental.pallas import tpu as pltpu

_DEPTH = 2


def _chunk_table(B, N):
    table = []
    for b in range(B):
        if b == 0:
            sizes = [N // 8, N // 8, N // 4, N // 2]
        elif b == B - 1:
            sizes = [N // 2, N // 4, N // 4]
        else:
            sizes = [N // 2, N // 2]
        s = 0
        for rows in sizes:
            table.append((b * N + s, s, rows))
            s += rows
    return table


def _body_factory(B, N, D):
    table = _chunk_table(B, N)
    pos_chunks = [(ps, rows) for (fs, ps, rows) in table if fs < N]

    def body(x_hbm, pos_hbm, o_hbm, x_buf, o_buf, pos_vmem, in_sems, out_sems, pos_sems):
        def in_cp(i, k):
            fs, _, rows = table[i]
            return pltpu.make_async_copy(
                x_hbm.at[pl.ds(fs, rows)], x_buf.at[k, pl.ds(0, rows)],
                in_sems.at[k])

        def out_cp(i, k):
            fs, _, rows = table[i]
            return pltpu.make_async_copy(
                o_buf.at[k, pl.ds(0, rows)], o_hbm.at[pl.ds(fs, rows)],
                out_sems.at[k])

        def pos_cp(j):
            ps, rows = pos_chunks[j]
            return pltpu.make_async_copy(
                pos_hbm.at[pl.ds(ps, rows)], pos_vmem.at[pl.ds(ps, rows)],
                pos_sems.at[j])

        pos_cp(0).start()
        for k in range(_DEPTH):
            in_cp(k, k).start()
        for j in range(1, len(pos_chunks)):
            pos_cp(j).start()

        n_chunks = len(table)
        for i in range(n_chunks):
            k = i % _DEPTH
            _, ps, rows = table[i]
            if i < len(pos_chunks):
                pos_cp(i).wait()
            in_cp(i, k).wait()
            if i >= _DEPTH:
                out_cp(i - _DEPTH, k).wait()
            o_buf[k, pl.ds(0, rows)] = (
                x_buf[k, pl.ds(0, rows)] + pos_vmem[pl.ds(ps, rows), :]
            )
            out_cp(i, k).start()
            if i + _DEPTH < n_chunks:
                in_cp(i + _DEPTH, k).start()

        for i in range(n_chunks - _DEPTH, n_chunks):
            out_cp(i, i % _DEPTH).wait()

    return body


@jax.jit
def kernel(x, token_embedding):
    B, N, D = x.shape
    out = pl.pallas_call(
        _body_factory(B, N, D),
        in_specs=[
            pl.BlockSpec(memory_space=pltpu.HBM),
            pl.BlockSpec(memory_space=pltpu.HBM),
        ],
        out_specs=pl.BlockSpec(memory_space=pltpu.HBM),
        out_shape=jax.ShapeDtypeStruct((B * N, D), x.dtype),
        scratch_shapes=[
            pltpu.VMEM((_DEPTH, N // 2, D), x.dtype),
            pltpu.VMEM((_DEPTH, N // 2, D), x.dtype),
            pltpu.VMEM((N, D), x.dtype),
            pltpu.SemaphoreType.DMA((_DEPTH,)),
            pltpu.SemaphoreType.DMA((_DEPTH,)),
            pltpu.SemaphoreType.DMA((sum(1 for (fs, _, _) in _chunk_table(B, N) if fs < N),)),
        ],
    )(x.reshape(B * N, D), token_embedding)
    return out.reshape(B, N, D)
